# baseline (speedup 1.0000x reference)
import functools

import jax
import jax.numpy as jnp
from jax import lax
from jax.experimental import pallas as pl
from jax.experimental.pallas import tpu as pltpu
from jax.experimental.pallas import tpu_sc as plsc

B = 16384
C = 5
V = 1000000
D = 64

NC = 2
NS = 16
NW = NC * NS
BPW = B // NW
CBT = 16
NCHUNK = BPW // CBT
XPC = CBT * C

TCW = 512
NBLK = (V + TCW - 1) // TCW
STC_ROWS = NBLK * TCW // 2


def _tc_transpose_body(x_ref, o_ref):
    x = x_ref[...]
    t = x.T
    o_ref[...] = jnp.concatenate([t[0:TCW // 2], t[TCW // 2:]], axis=1)[None]


_tc_transpose = pl.pallas_call(
    _tc_transpose_body,
    grid=(NBLK,),
    in_specs=[pl.BlockSpec((D, TCW), lambda i: (0, i))],
    out_specs=pl.BlockSpec((1, TCW // 2, 2 * D), lambda i: (i, 0, 0)),
    out_shape=jax.ShapeDtypeStruct((NBLK, TCW // 2, 2 * D), jnp.float32),
)


def _sc_body(tq_hbm, tr_hbm, xq_hbm, st_hbm, ct_hbm, out_hbm,
             tq_v, tr_v, xq_v, out_v, tt0, xt0, tt1, xt1, sem_t, sem_x):
    w = lax.axis_index("s") * NC + lax.axis_index("c")
    lane = lax.iota(jnp.int32, 16)
    pltpu.sync_copy(tq_hbm.at[pl.ds(w * BPW, BPW)], tq_v)
    pltpu.sync_copy(tr_hbm.at[pl.ds(w * BPW, BPW)], tr_v)
    pltpu.sync_copy(xq_hbm.at[pl.ds(w * BPW * C, BPW * C)], xq_v)

    tbufs = (tt0, tt1)
    xbufs = (xt0, xt1)

    def fire(q, tbuf, xbuf):
        pltpu.async_copy(st_hbm.at[tq_v.at[pl.ds(q * CBT, CBT)]], tbuf, sem_t)
        pltpu.async_copy(ct_hbm.at[xq_v.at[pl.ds(q * XPC, XPC)]], xbuf, sem_x)

    def drain(tbuf, xbuf):
        pltpu.make_async_copy(st_hbm.at[pl.ds(0, CBT)], tbuf, sem_t).wait()
        pltpu.make_async_copy(ct_hbm.at[pl.ds(0, XPC)], xbuf, sem_x).wait()

    fire(jnp.int32(0), tbufs[0], xbufs[0])

    def outer(ob, carry):
        for par in range(2):
            q = ob * 2 + par
            fire(jnp.minimum(q + 1, NCHUNK - 1),
                 tbufs[(par + 1) % 2], xbufs[(par + 1) % 2])
            drain(tbufs[par], xbufs[par])
            tbuf = tbufs[par]
            xbuf = xbufs[par]
            for g in range(C):
                pp = g * 16 + lane
                trow = pp // C
                tb = plsc.load_gather(tr_v, [q * CBT + trow])
                acc0 = jnp.zeros((16,), jnp.float32)
                acc1 = jnp.zeros((16,), jnp.float32)

                def dstep(k, accs):
                    a0, a1 = accs
                    d0 = k * 2
                    dv0 = jnp.full((16,), d0, jnp.int32)
                    dv1 = jnp.full((16,), d0 + 1, jnp.int32)
                    t0 = plsc.load_gather(tbuf, [trow, tb + dv0])
                    x0 = plsc.load_gather(xbuf, [pp, dv0])
                    t1 = plsc.load_gather(tbuf, [trow, tb + dv1])
                    x1 = plsc.load_gather(xbuf, [pp, dv1])
                    return a0 + t0 * x0, a1 + t1 * x1

                acc0, acc1 = lax.fori_loop(0, D // 2, dstep, (acc0, acc1),
                                           unroll=4)
                out_v[pl.ds(q * XPC + g * 16, 16)] = acc0 + acc1
        return carry

    lax.fori_loop(0, NCHUNK // 2, outer, 0)
    drain(tbufs[0], xbufs[0])
    pltpu.sync_copy(out_v, out_hbm.at[pl.ds(w * BPW * C, BPW * C)])


_mesh = plsc.VectorSubcoreMesh(core_axis_name="c", subcore_axis_name="s")

_skipgram_sc = functools.partial(
    pl.kernel,
    out_type=jax.ShapeDtypeStruct((B * C,), jnp.float32),
    mesh=_mesh,
    scratch_types=[
        pltpu.VMEM((BPW,), jnp.int32),
        pltpu.VMEM((BPW,), jnp.int32),
        pltpu.VMEM((BPW * C,), jnp.int32),
        pltpu.VMEM((BPW * C,), jnp.float32),
        pltpu.VMEM((CBT, 2 * D), jnp.float32),
        pltpu.VMEM((XPC, D), jnp.float32),
        pltpu.VMEM((CBT, 2 * D), jnp.float32),
        pltpu.VMEM((XPC, D), jnp.float32),
        pltpu.SemaphoreType.DMA,
        pltpu.SemaphoreType.DMA,
    ],
    compiler_params=pltpu.CompilerParams(
        needs_layout_passes=False, use_tc_tiling_on_sc=False),
)(_sc_body)


def kernel(target, context, skipgram_table, context_table):
    tgt = target.astype(jnp.int32)
    ctx = context.astype(jnp.int32).reshape(B * C)
    st_c = _tc_transpose(skipgram_table.T).reshape(STC_ROWS, 2 * D)
    tq = ((tgt >> 9) << 8) | (tgt & 255)
    tr = ((tgt >> 8) & 1) << 6
    out = _skipgram_sc(tq, tr, ctx, st_c, context_table)
    return out.reshape(B, C)


# speedup vs baseline: 1.6796x; 1.6796x over previous
import functools

import jax
import jax.numpy as jnp
from jax import lax
from jax.experimental import pallas as pl
from jax.experimental.pallas import tpu as pltpu
from jax.experimental.pallas import tpu_sc as plsc

B = 16384
C = 5
V = 1000000
D = 64

NC = 2
NS = 16
NW = NC * NS
BPW = B // NW
CBT = 16
NCHUNK = BPW // CBT
XPC = CBT * C

TCW = 2048
NBLK = (V + TCW - 1) // TCW
STC_ROWS = NBLK * TCW // 2


def _tc_transpose_body(x_ref, o_ref):
    x = x_ref[...]
    eye = jnp.eye(D, dtype=jnp.float32)
    t = lax.dot_general(x, eye, (((0,), (0,)), ((), ())),
                        preferred_element_type=jnp.float32)
    o_ref[...] = jnp.concatenate([t[0:TCW // 2], t[TCW // 2:]], axis=1)[None]


_tc_transpose = pl.pallas_call(
    _tc_transpose_body,
    grid=(NBLK,),
    in_specs=[pl.BlockSpec((D, TCW), lambda i: (0, i))],
    out_specs=pl.BlockSpec((1, TCW // 2, 2 * D), lambda i: (i, 0, 0)),
    out_shape=jax.ShapeDtypeStruct((NBLK, TCW // 2, 2 * D), jnp.float32),
)


def _sc_body(tq_hbm, tr_hbm, xq_hbm, st_hbm, ct_hbm, out_hbm,
             tq_v, tr_v, xq_v, out_v, tt0, xt0, tt1, xt1, sem_t, sem_x):
    w = lax.axis_index("s") * NC + lax.axis_index("c")
    lane = lax.iota(jnp.int32, 16)
    pltpu.sync_copy(tq_hbm.at[pl.ds(w * BPW, BPW)], tq_v)
    pltpu.sync_copy(tr_hbm.at[pl.ds(w * BPW, BPW)], tr_v)
    pltpu.sync_copy(xq_hbm.at[pl.ds(w * BPW * C, BPW * C)], xq_v)

    tbufs = (tt0, tt1)
    xbufs = (xt0, xt1)

    def fire(q, tbuf, xbuf):
        pltpu.async_copy(st_hbm.at[tq_v.at[pl.ds(q * CBT, CBT)]], tbuf, sem_t)
        pltpu.async_copy(ct_hbm.at[xq_v.at[pl.ds(q * XPC, XPC)]], xbuf, sem_x)

    def drain(tbuf, xbuf):
        pltpu.make_async_copy(st_hbm.at[pl.ds(0, CBT)], tbuf, sem_t).wait()
        pltpu.make_async_copy(ct_hbm.at[pl.ds(0, XPC)], xbuf, sem_x).wait()

    fire(jnp.int32(0), tbufs[0], xbufs[0])

    def outer(ob, carry):
        for par in range(2):
            q = ob * 2 + par
            fire(jnp.minimum(q + 1, NCHUNK - 1),
                 tbufs[(par + 1) % 2], xbufs[(par + 1) % 2])
            drain(tbufs[par], xbufs[par])
            tbuf = tbufs[par]
            xbuf = xbufs[par]
            for g in range(C):
                pp = g * 16 + lane
                trow = pp // C
                tb = plsc.load_gather(tr_v, [q * CBT + trow])
                acc0 = jnp.zeros((16,), jnp.float32)
                acc1 = jnp.zeros((16,), jnp.float32)

                def dstep(k, accs):
                    a0, a1 = accs
                    d0 = k * 2
                    dv0 = jnp.full((16,), d0, jnp.int32)
                    dv1 = jnp.full((16,), d0 + 1, jnp.int32)
                    t0 = plsc.load_gather(tbuf, [trow, tb + dv0])
                    x0 = plsc.load_gather(xbuf, [pp, dv0])
                    t1 = plsc.load_gather(tbuf, [trow, tb + dv1])
                    x1 = plsc.load_gather(xbuf, [pp, dv1])
                    return a0 + t0 * x0, a1 + t1 * x1

                acc0, acc1 = lax.fori_loop(0, D // 2, dstep, (acc0, acc1),
                                           unroll=4)
                out_v[pl.ds(q * XPC + g * 16, 16)] = acc0 + acc1
        return carry

    lax.fori_loop(0, NCHUNK // 2, outer, 0)
    drain(tbufs[0], xbufs[0])
    pltpu.sync_copy(out_v, out_hbm.at[pl.ds(w * BPW * C, BPW * C)])


_mesh = plsc.VectorSubcoreMesh(core_axis_name="c", subcore_axis_name="s")

_skipgram_sc = functools.partial(
    pl.kernel,
    out_type=jax.ShapeDtypeStruct((B * C,), jnp.float32),
    mesh=_mesh,
    scratch_types=[
        pltpu.VMEM((BPW,), jnp.int32),
        pltpu.VMEM((BPW,), jnp.int32),
        pltpu.VMEM((BPW * C,), jnp.int32),
        pltpu.VMEM((BPW * C,), jnp.float32),
        pltpu.VMEM((CBT, 2 * D), jnp.float32),
        pltpu.VMEM((XPC, D), jnp.float32),
        pltpu.VMEM((CBT, 2 * D), jnp.float32),
        pltpu.VMEM((XPC, D), jnp.float32),
        pltpu.SemaphoreType.DMA,
        pltpu.SemaphoreType.DMA,
    ],
    compiler_params=pltpu.CompilerParams(
        needs_layout_passes=False, use_tc_tiling_on_sc=False),
)(_sc_body)


def kernel(target, context, skipgram_table, context_table):
    tgt = target.astype(jnp.int32)
    ctx = context.astype(jnp.int32).reshape(B * C)
    st_c = _tc_transpose(skipgram_table.T).reshape(STC_ROWS, 2 * D)
    tq = ((tgt >> 11) << 10) | (tgt & 1023)
    tr = ((tgt >> 10) & 1) << 6
    out = _skipgram_sc(tq, tr, ctx, st_c, context_table)
    return out.reshape(B, C)
